# baseline (speedup 1.0000x reference)
import functools

import jax
import jax.numpy as jnp
from jax import lax
from jax.experimental import pallas as pl
from jax.experimental.pallas import tpu as pltpu
from jax.experimental.pallas import tpu_sc as plsc

P_ = 24
H_ = 12
W_ = 25
HL_ = 16
PAIR_ = 32
SCYC_ = 16


def _pass1(S, xbuf, ls, tbuf):
    n_cycles = S // P_

    w = xbuf[0, ls]
    for d in range(1, H_ + 1):
        w = w + xbuf[d, ls]

    psum = [None] * P_
    for t in range(P_):
        r = 1.0 / (H_ + 1 + t) if t <= H_ else 1.0 / W_
        tr = w * r
        tbuf[t, :] = tr
        psum[t] = xbuf[t, ls] - tr
        w = w + xbuf[t + H_ + 1, ls]
        if t >= H_:
            w = w - xbuf[t - H_, ls]

    def mid_body(c, carry):
        wc = carry[0]
        ps = list(carry[1:])
        base = c * P_
        for p in range(P_):
            t = base + p
            tr = wc * (1.0 / W_)
            tbuf[t, :] = tr
            ps[p] = ps[p] + (xbuf[t, ls] - tr)
            wc = wc + xbuf[t + H_ + 1, ls] - xbuf[t - H_, ls]
        return (wc, *ps)

    carry = lax.fori_loop(1, n_cycles - 1, mid_body, (w, *psum),
                          unroll=False)
    w = carry[0]
    psum = list(carry[1:])

    for t in range((n_cycles - 1) * P_, S):
        p = t % P_
        r = 1.0 / W_ if t + H_ + 1 <= S else 1.0 / (S - t + H_)
        tr = w * r
        tbuf[t, :] = tr
        psum[p] = psum[p] + (xbuf[t, ls] - tr)
        if t + H_ + 1 < S:
            w = w + xbuf[t + H_ + 1, ls]
        w = w - xbuf[t - H_, ls]
    return psum


def _pass2(S, xbuf, ls, tbuf, sbuf, pat):
    n_cycles = S // P_

    def p2a_body(c, dummy):
        base = c * P_
        for p in range(P_):
            t = base + p
            sbuf[t, ls] = pat[p]
            xbuf[t, ls] = xbuf[t, ls] - tbuf[t, :] - pat[p]
        return dummy

    def p2b_body(c, dummy):
        base = c * P_
        for p in range(P_):
            t = base + p
            xbuf[t, ls] = xbuf[t, ls] - tbuf[t, :] - pat[p]
        return dummy

    lax.fori_loop(0, SCYC_, p2a_body, jnp.int32(0), unroll=False)
    lax.fori_loop(SCYC_, n_cycles, p2b_body, jnp.int32(0), unroll=False)
    for t in range(n_cycles * P_, S):
        p = t % P_
        xbuf[t, ls] = xbuf[t, ls] - tbuf[t, :] - pat[p]


def _decomp_body(S, B, F, n_pairs, x_hbm, trend_hbm, seasonal_hbm,
                 residual_hbm, xbuf, tbuf, sbuf,
                 sem_in, sem_t, sem_s, sem_r):
    n_cycles = S // P_
    rem = S % P_
    info = plsc.get_sparse_core_info()
    nc = info.num_cores
    groups = F // HL_
    wid = lax.axis_index("s") * nc + lax.axis_index("c")
    n_tasks = n_pairs * 2

    def loc(q):
        task = wid * n_tasks + 2 * q
        return task // groups, (task % groups) * HL_

    t_h = None
    s_h = []
    r_h = None

    b0, l0 = loc(0)
    in_h = pltpu.async_copy(x_hbm.at[b0, :, pl.ds(l0, PAIR_)], xbuf, sem_in)

    for q in range(n_pairs):
        b, l = loc(q)
        in_h.wait()
        for u in range(2):
            ls = slice(u * HL_, u * HL_ + HL_)
            if t_h is not None:
                t_h.wait()
            psum = _pass1(S, xbuf, ls, tbuf)
            t_h = pltpu.async_copy(
                tbuf, trend_hbm.at[b, :, pl.ds(l + u * HL_, HL_)], sem_t)
            pat = [psum[p] * (1.0 / (n_cycles + 1 if p < rem else n_cycles))
                   for p in range(P_)]
            if u == 0 and s_h:
                for c in s_h:
                    c.wait()
                s_h = []
            _pass2(S, xbuf, ls, tbuf, sbuf, pat)

        r_h = pltpu.async_copy(
            xbuf, residual_hbm.at[b, :, pl.ds(l, PAIR_)], sem_r)
        n_sr = SCYC_ * P_
        off = 0
        while off < S:
            rows = min(n_sr, S - off)
            s_h.append(pltpu.async_copy(
                sbuf.at[pl.ds(0, rows)],
                seasonal_hbm.at[b, pl.ds(off, rows), pl.ds(l, PAIR_)],
                sem_s))
            off += rows
        if q + 1 < n_pairs:
            nb, nl = loc(q + 1)
            r_h.wait()
            r_h = None
            in_h = pltpu.async_copy(x_hbm.at[nb, :, pl.ds(nl, PAIR_)],
                                    xbuf, sem_in)

    t_h.wait()
    for c in s_h:
        c.wait()
    if r_h is not None:
        r_h.wait()


@jax.jit
def _decompose(x):
    B, S, F = x.shape
    info = plsc.get_sparse_core_info()
    n_workers = info.num_cores * info.num_subcores
    n_tasks = B * (F // HL_)
    assert n_tasks % (n_workers * 2) == 0
    mesh = plsc.VectorSubcoreMesh(core_axis_name="c", subcore_axis_name="s")
    out = jax.ShapeDtypeStruct((B, S, F), x.dtype)
    body = functools.partial(_decomp_body, S, B, F,
                             n_tasks // n_workers // 2)
    return pl.kernel(
        body,
        out_type=(out, out, out),
        mesh=mesh,
        scratch_types=[
            pltpu.VMEM((S, PAIR_), jnp.float32),
            pltpu.VMEM((S, HL_), jnp.float32),
            pltpu.VMEM((SCYC_ * P_, PAIR_), jnp.float32),
            pltpu.SemaphoreType.DMA,
            pltpu.SemaphoreType.DMA,
            pltpu.SemaphoreType.DMA,
            pltpu.SemaphoreType.DMA,
        ],
        compiler_params=pltpu.CompilerParams(use_tc_tiling_on_sc=False),
    )(x)


def kernel(x):
    trend, seasonal, residual = _decompose(x)
    return (trend, seasonal, residual, x)
